# Initial kernel scaffold; baseline (speedup 1.0000x reference)
import jax
import jax.numpy as jnp
from jax.experimental import pallas as pl


def kernel(data, table, w, bias):
    raise NotImplementedError("write your pallas kernel here")


# speedup vs baseline: 1.2605x; 1.2605x over previous
import functools

import jax
import jax.numpy as jnp
from jax import lax
from jax.experimental import pallas as pl
from jax.experimental.pallas import tpu as pltpu
from jax.experimental.pallas import tpu_sc as plsc

def _lane_sum(v):
    iota = jnp.arange(16, dtype=jnp.int32)
    for sh in (8, 4, 2, 1):
        v = v + v.at[iota ^ sh].get(mode="promise_in_bounds")
    return v


F = 26
D = 16
B = 16384
VOCAB = 40000
C = 64
G = C * F // 128
NC = 2
NS = 16


def _fm_kernel(data_hbm, table_hbm, w_hbm, bias_hbm, out_hbm,
               data_v, widx_v, grows_v, wrow_v, fm_v, out_v, bias_v, sem):
    chunks_per_w = B // C // (NC * NS)
    wid = lax.axis_index("s") * NC + lax.axis_index("c")

    bias_v[...] = jnp.zeros((16,), jnp.float32)
    pltpu.sync_copy(bias_hbm, bias_v.at[pl.ds(0, 1)])
    bias_s = _lane_sum(bias_v[...])

    def chunk_body(t, carry):
        tg = wid * chunks_per_w + t
        pltpu.sync_copy(data_hbm.at[tg], data_v)

        for f in range(F):
            off = f * VOCAB
            for j in range(C // 16):
                p = f * C + j * 16
                widx_v[p // 128, pl.ds(p % 128, 16)] = (
                    data_v[f, pl.ds(j * 16, 16)] + off)

        descs = []
        for g in range(G):
            descs.append(pltpu.async_copy(
                table_hbm.at[widx_v.at[g]],
                grows_v.at[pl.ds(g * 128, 128)], sem))
        for g in range(G):
            descs.append(pltpu.async_copy(
                w_hbm.at[widx_v.at[g]],
                wrow_v.at[pl.ds(g * 128, 128)], sem))
        for dsc in descs:
            dsc.wait()

        lane0 = jnp.arange(16, dtype=jnp.int32) == 0
        zi = jnp.zeros((16,), jnp.int32)

        def row_body(b, carry2):
            s = grows_v[b]
            ss = s * s
            for f in range(1, F):
                v = grows_v[f * C + b]
                s = s + v
                ss = ss + v * v
            fm = 0.5 * _lane_sum(s * s - ss)
            plsc.store_scatter(fm_v, [zi + b], fm, mask=lane0)
            return carry2

        lax.fori_loop(0, C, row_body, 0)

        for j in range(C // 16):
            acc = wrow_v[pl.ds(j * 16, 16)]
            for f in range(1, F):
                acc = acc + wrow_v[pl.ds(f * C + j * 16, 16)]
            x = acc + bias_s + fm_v[pl.ds(j * 16, 16)]
            out_v[pl.ds(j * 16, 16)] = 1.0 / (1.0 + jnp.exp(-x))
        pltpu.sync_copy(out_v, out_hbm.at[pl.ds(tg * C, C)])
        return carry

    lax.fori_loop(0, chunks_per_w, chunk_body, 0)


def kernel(data, table, w, bias):
    data3 = data.reshape(B // C, C, F).transpose(0, 2, 1)
    w1 = w.reshape(-1)

    mesh = plsc.VectorSubcoreMesh(core_axis_name="c", subcore_axis_name="s",
                                  num_cores=NC, num_subcores=NS)
    run = pl.kernel(
        _fm_kernel,
        out_type=jax.ShapeDtypeStruct((B,), jnp.float32),
        mesh=mesh,
        compiler_params=pltpu.CompilerParams(
            needs_layout_passes=False, use_tc_tiling_on_sc=False),
        scratch_types=[
            pltpu.VMEM((F, C), jnp.int32),
            pltpu.VMEM((G, 128), jnp.int32),
            pltpu.VMEM((F * C, D), jnp.float32),
            pltpu.VMEM((F * C,), jnp.float32),
            pltpu.VMEM((C,), jnp.float32),
            pltpu.VMEM((C,), jnp.float32),
            pltpu.VMEM((16,), jnp.float32),
            pltpu.SemaphoreType.DMA,
        ],
    )
    return run(data3, table, w1, bias)
